# baseline (speedup 1.0000x reference)
import functools

import jax
import jax.numpy as jnp
from jax import lax
from jax.experimental import pallas as pl
from jax.experimental.pallas import tpu as pltpu
from jax.experimental.pallas import tpu_sc as plsc

N = 10000
K = 32
NE = N * K
H = 128
XE = 128

NW = 32
NPW = 320
CH = 64
NCHUNK = NPW // CH
N_PAD = NW * NPW
NE_PAD = N_PAD * K
BLK = 128
SUB = BLK // 16
NB = CH * K // BLK
NBUF = 2


def _sc_body(h_hbm, c_hbm, type_hbm, src_hbm, zeros_hbm,
             mail_hbm, cout_hbm, tout_hbm,
             typev, idx_all, tbuf_all, didx, idxb, hrows, crows, accsh,
             semg, semsc, semcw):
    sid = lax.axis_index("s")
    wid = lax.axis_index("c") * 16 + sid
    abase = sid * (2 * CH)
    pltpu.sync_copy(type_hbm, typev)
    iota16 = lax.iota(jnp.int32, 16)

    def _chunk(ci, carry):
        node_start = wid * NPW + ci * CH
        e0c = node_start * K
        pltpu.sync_copy(zeros_hbm, accsh.at[pl.ds(abase, 2 * CH)])

        def _blk(bi, bcarry):
            e0 = e0c + bi * BLK
            pltpu.sync_copy(src_hbm.at[pl.ds(e0, BLK)], idxb[0])
            for j in range(SUB):
                src16 = idxb[0][pl.ds(j * 16, 16)]
                r16 = lax.shift_right_logical(src16, 7)
                c16 = jnp.bitwise_and(src16, 127)
                t16 = plsc.load_gather(typev, [r16, c16])
                tbuf_all[pl.ds(j * 16, 16)] = t16
                nl = lax.shift_right_logical(bi * BLK + j * 16 + iota16, 5)
                didx[0][pl.ds(j * 16, 16)] = abase + nl * 2 + t16
            cph = pltpu.async_copy(h_hbm.at[idxb[0]], hrows[0], semg[0])
            cpc = pltpu.async_copy(c_hbm.at[idxb[0]], crows[0], semg[0])
            cph.wait()
            pltpu.sync_copy(hrows[0], accsh.at[didx[0]], add=True)
            cpc.wait()
            pltpu.sync_copy(crows[0], cout_hbm.at[pl.ds(e0, BLK)])
            pltpu.sync_copy(tbuf_all.at[pl.ds(0, BLK)],
                            tout_hbm.at[pl.ds(e0, BLK)])
            return bcarry

        lax.fori_loop(0, NB, _blk, jnp.int32(0))
        pltpu.sync_copy(accsh.at[pl.ds(abase, 2 * CH)],
                        mail_hbm.at[pl.ds(node_start * 2, 2 * CH)])
        return carry

    lax.fori_loop(0, NCHUNK, _chunk, jnp.int32(0))


_sc_gather = pl.kernel(
    _sc_body,
    out_type=(
        jax.ShapeDtypeStruct((2 * N_PAD, H), jnp.float32),
        jax.ShapeDtypeStruct((NE_PAD, H), jnp.float32),
        jax.ShapeDtypeStruct((NE_PAD,), jnp.int32),
    ),
    mesh=plsc.VectorSubcoreMesh(core_axis_name="c", subcore_axis_name="s"),
    scratch_types=[
        pltpu.VMEM((80, 128), jnp.int32),
        pltpu.VMEM((CH * K,), jnp.int32),
        pltpu.VMEM((CH * K,), jnp.int32),
        [pltpu.VMEM((BLK,), jnp.int32) for _ in range(NBUF)],
        [pltpu.VMEM((BLK,), jnp.int32) for _ in range(NBUF)],
        [pltpu.VMEM((BLK, H), jnp.float32) for _ in range(NBUF)],
        [pltpu.VMEM((BLK, H), jnp.float32) for _ in range(NBUF)],
        pltpu.VMEM_SHARED((16 * 2 * CH, H), jnp.float32),
        [pltpu.SemaphoreType.DMA for _ in range(NBUF)],
        [pltpu.SemaphoreType.DMA for _ in range(NBUF)],
        [pltpu.SemaphoreType.DMA for _ in range(NBUF)],
    ],
    compiler_params=pltpu.CompilerParams(needs_layout_passes=False),
)

BN = 400
BE = BN * K
GRID = N // BN


def _tc_body(emb3_ref, mail_ref, crows_ref, t_ref,
             W_iou_ref, U_iou_ref, b_iou_ref, W_f_ref, U_f_ref, U_f_b_ref,
             b_f_ref, h_out, c_out):
    i = pl.program_id(0)

    def matT(x, w):
        return lax.dot_general(x, w, (((1,), (1,)), ((), ())),
                               preferred_element_type=jnp.float32)

    h_iou = mail_ref[...]
    f = matT(h_iou, U_f_ref[...]) + U_f_b_ref[...]
    b_f = b_f_ref[...]
    f0 = f[:, :H] + b_f
    f1 = f[:, H:] + b_f

    emb_blk = emb3_ref[pl.ds(i * BN, BN), :]
    iou = (matT(emb_blk, W_iou_ref[...]) + matT(h_iou, U_iou_ref[...])
           + b_iou_ref[...])

    s = (i * BE) % N
    embe = emb3_ref[pl.ds(s, BE), :]
    Xe = matT(embe, W_f_ref[...]).reshape(BN, K, H)
    tb = lax.broadcast_in_dim(t_ref[...].astype(jnp.float32),
                              (BN, K, H), (0, 1))
    f0b = lax.broadcast_in_dim(f0, (BN, K, H), (0, 2))
    dfb = lax.broadcast_in_dim(f1 - f0, (BN, K, H), (0, 2))
    w = jax.nn.sigmoid(Xe + f0b + tb * dfb)
    c_cell = jnp.sum(w * crows_ref[...].reshape(BN, K, H), axis=1)

    ig = jax.nn.sigmoid(iou[:, :H])
    og = jax.nn.sigmoid(iou[:, H:2 * H])
    ug = jnp.tanh(iou[:, 2 * H:])
    c_new = ig * ug + c_cell
    h_out[...] = og * jnp.tanh(c_new)
    c_out[...] = c_new


_tc_dense = pl.pallas_call(
    _tc_body,
    grid=(GRID,),
    in_specs=[
        pl.BlockSpec((3 * N, XE), lambda i: (0, 0)),
        pl.BlockSpec((BN, 2 * H), lambda i: (i, 0)),
        pl.BlockSpec((BE, H), lambda i: (i, 0)),
        pl.BlockSpec((BN, K), lambda i: (i, 0)),
        pl.BlockSpec((3 * H, XE), lambda i: (0, 0)),
        pl.BlockSpec((3 * H, 2 * H), lambda i: (0, 0)),
        pl.BlockSpec((1, 3 * H), lambda i: (0, 0)),
        pl.BlockSpec((H, XE), lambda i: (0, 0)),
        pl.BlockSpec((2 * H, 2 * H), lambda i: (0, 0)),
        pl.BlockSpec((1, 2 * H), lambda i: (0, 0)),
        pl.BlockSpec((1, H), lambda i: (0, 0)),
    ],
    out_specs=[
        pl.BlockSpec((BN, H), lambda i: (i, 0)),
        pl.BlockSpec((BN, H), lambda i: (i, 0)),
    ],
    out_shape=[
        jax.ShapeDtypeStruct((N, H), jnp.float32),
        jax.ShapeDtypeStruct((N, H), jnp.float32),
    ],
)


def kernel(emb, h, c, type_n, edge_index, W_iou_w, U_iou_w, b_iou, W_f_w,
           U_f_w, U_f_b, b_f):
    src = edge_index[0]
    src_pad = jnp.concatenate(
        [src, jnp.zeros((NE_PAD - NE,), src.dtype)])
    zeros = jnp.zeros((2 * CH, H), jnp.float32)
    type_pad = jnp.concatenate(
        [type_n, jnp.zeros((80 * 128 - N,), jnp.int32)]).reshape(80, 128)
    mail, c_rows, t_child = _sc_gather(h, c, type_pad, src_pad, zeros)
    mail2 = mail.reshape(N_PAD, 2 * H)
    t_nk = t_child[:NE].reshape(N, K)
    emb3 = jnp.concatenate([emb, emb, emb], axis=0)
    h_new, c_new = _tc_dense(emb3, mail2, c_rows, t_nk,
                             W_iou_w, U_iou_w, b_iou.reshape(1, 3 * H),
                             W_f_w, U_f_w, U_f_b.reshape(1, 2 * H),
                             b_f.reshape(1, H))
    return (h_new, c_new)


# speedup vs baseline: 1.9549x; 1.9549x over previous
import functools

import jax
import jax.numpy as jnp
from jax import lax
from jax.experimental import pallas as pl
from jax.experimental.pallas import tpu as pltpu
from jax.experimental.pallas import tpu_sc as plsc

N = 10000
K = 32
NE = N * K
H = 128
XE = 128

NW = 32
NPW = 320
CH = 64
NCHUNK = NPW // CH
N_PAD = NW * NPW
NE_PAD = N_PAD * K
BLK = 128
SUB = BLK // 16
NB = CH * K // BLK
NBUF = 2


def _sc_body(h_hbm, c_hbm, type_hbm, src_hbm, zeros_hbm,
             mail_hbm, cout_hbm, tout_hbm,
             typev, idx_v, didx, tbuf, hrows, crows, accsh, semh, semc):
    sid = lax.axis_index("s")
    wid = lax.axis_index("c") * 16 + sid
    abase = sid * (2 * CH)
    pltpu.sync_copy(type_hbm, typev)
    for ci in range(NPW // CH):
        node_start = wid * NPW + ci * CH
        @pl.when(node_start < N)
        def _chunk():
            valid = jnp.minimum(jnp.int32(CH), jnp.int32(N) - node_start)
            nblk = (valid * K) // BLK
            pltpu.sync_copy(zeros_hbm, accsh.at[pl.ds(abase, 2 * CH)])

            def _blk(bi, carry):
                e0 = node_start * K + bi * BLK
                pltpu.sync_copy(src_hbm.at[pl.ds(e0, BLK)], idx_v)
                for j in range(SUB):
                    src16 = idx_v[pl.ds(j * 16, 16)]
                    r16 = lax.shift_right_logical(src16, 7)
                    c16 = jnp.bitwise_and(src16, 127)
                    t16 = plsc.load_gather(typev, [r16, c16])
                    tbuf[pl.ds(j * 16, 16)] = t16
                    eloc = bi * BLK + j * 16 + lax.iota(jnp.int32, 16)
                    nl = lax.shift_right_logical(eloc, 5)
                    didx[pl.ds(j * 16, 16)] = abase + nl * 2 + t16
                cph = pltpu.async_copy(h_hbm.at[idx_v], hrows, semh)
                cpc = pltpu.async_copy(c_hbm.at[idx_v], crows, semc)
                cph.wait()
                pltpu.sync_copy(hrows, accsh.at[didx], add=True)
                cpc.wait()
                pltpu.sync_copy(crows, cout_hbm.at[pl.ds(e0, BLK)])
                pltpu.sync_copy(tbuf, tout_hbm.at[pl.ds(e0, BLK)])
                return carry

            lax.fori_loop(0, nblk, _blk, jnp.int32(0))
            pltpu.sync_copy(accsh.at[pl.ds(abase, 2 * CH)],
                            mail_hbm.at[pl.ds(node_start * 2, 2 * CH)])


_sc_gather = pl.kernel(
    _sc_body,
    out_type=(
        jax.ShapeDtypeStruct((2 * N_PAD, H), jnp.float32),
        jax.ShapeDtypeStruct((NE, H), jnp.float32),
        jax.ShapeDtypeStruct((NE,), jnp.int32),
    ),
    mesh=plsc.VectorSubcoreMesh(core_axis_name="c", subcore_axis_name="s"),
    scratch_types=[
        pltpu.VMEM((80, 128), jnp.int32),
        pltpu.VMEM((BLK,), jnp.int32),
        pltpu.VMEM((BLK,), jnp.int32),
        pltpu.VMEM((BLK,), jnp.int32),
        pltpu.VMEM((BLK, H), jnp.float32),
        pltpu.VMEM((BLK, H), jnp.float32),
        pltpu.VMEM_SHARED((16 * 2 * CH, H), jnp.float32),
        pltpu.SemaphoreType.DMA,
        pltpu.SemaphoreType.DMA,
    ],
    compiler_params=pltpu.CompilerParams(needs_layout_passes=False),
)

BN = 400
BE = BN * K
GRID = N // BN


def _tc_body(emb3_ref, mail_ref, crows_ref, t_ref,
             W_iou_ref, U_iou_ref, b_iou_ref, W_f_ref, U_f_ref, U_f_b_ref,
             b_f_ref, h_out, c_out):
    i = pl.program_id(0)

    def matT(x, w):
        return lax.dot_general(x, w, (((1,), (1,)), ((), ())),
                               preferred_element_type=jnp.float32)

    h_iou = mail_ref[...]
    f = matT(h_iou, U_f_ref[...]) + U_f_b_ref[...]
    b_f = b_f_ref[...]
    f0 = f[:, :H] + b_f
    f1 = f[:, H:] + b_f

    emb_blk = emb3_ref[pl.ds(i * BN, BN), :]
    iou = (matT(emb_blk, W_iou_ref[...]) + matT(h_iou, U_iou_ref[...])
           + b_iou_ref[...])

    s = (i * BE) % N
    embe = emb3_ref[pl.ds(s, BE), :]
    Xe = matT(embe, W_f_ref[...]).reshape(BN, K, H)
    tb = lax.broadcast_in_dim(t_ref[...].astype(jnp.float32),
                              (BN, K, H), (0, 1))
    f0b = lax.broadcast_in_dim(f0, (BN, K, H), (0, 2))
    dfb = lax.broadcast_in_dim(f1 - f0, (BN, K, H), (0, 2))
    w = jax.nn.sigmoid(Xe + f0b + tb * dfb)
    c_cell = jnp.sum(w * crows_ref[...].reshape(BN, K, H), axis=1)

    ig = jax.nn.sigmoid(iou[:, :H])
    og = jax.nn.sigmoid(iou[:, H:2 * H])
    ug = jnp.tanh(iou[:, 2 * H:])
    c_new = ig * ug + c_cell
    h_out[...] = og * jnp.tanh(c_new)
    c_out[...] = c_new


_tc_dense = pl.pallas_call(
    _tc_body,
    grid=(GRID,),
    in_specs=[
        pl.BlockSpec((3 * N, XE), lambda i: (0, 0)),
        pl.BlockSpec((BN, 2 * H), lambda i: (i, 0)),
        pl.BlockSpec((BE, H), lambda i: (i, 0)),
        pl.BlockSpec((BN, K), lambda i: (i, 0)),
        pl.BlockSpec((3 * H, XE), lambda i: (0, 0)),
        pl.BlockSpec((3 * H, 2 * H), lambda i: (0, 0)),
        pl.BlockSpec((1, 3 * H), lambda i: (0, 0)),
        pl.BlockSpec((H, XE), lambda i: (0, 0)),
        pl.BlockSpec((2 * H, 2 * H), lambda i: (0, 0)),
        pl.BlockSpec((1, 2 * H), lambda i: (0, 0)),
        pl.BlockSpec((1, H), lambda i: (0, 0)),
    ],
    out_specs=[
        pl.BlockSpec((BN, H), lambda i: (i, 0)),
        pl.BlockSpec((BN, H), lambda i: (i, 0)),
    ],
    out_shape=[
        jax.ShapeDtypeStruct((N, H), jnp.float32),
        jax.ShapeDtypeStruct((N, H), jnp.float32),
    ],
)


def kernel(emb, h, c, type_n, edge_index, W_iou_w, U_iou_w, b_iou, W_f_w,
           U_f_w, U_f_b, b_f):
    src = edge_index[0]
    zeros = jnp.zeros((2 * CH, H), jnp.float32)
    type_pad = jnp.concatenate(
        [type_n, jnp.zeros((80 * 128 - N,), jnp.int32)]).reshape(80, 128)
    mail, c_rows, t_child = _sc_gather(h, c, type_pad, src, zeros)
    mail2 = mail.reshape(N_PAD, 2 * H)
    t_nk = t_child.reshape(N, K)
    emb3 = jnp.concatenate([emb, emb, emb], axis=0)
    h_new, c_new = _tc_dense(emb3, mail2, c_rows, t_nk,
                             W_iou_w, U_iou_w, b_iou.reshape(1, 3 * H),
                             W_f_w, U_f_w, U_f_b.reshape(1, 2 * H),
                             b_f.reshape(1, H))
    return (h_new, c_new)


# speedup vs baseline: 2.6631x; 1.3622x over previous
import functools

import jax
import jax.numpy as jnp
from jax import lax
from jax.experimental import pallas as pl
from jax.experimental.pallas import tpu as pltpu
from jax.experimental.pallas import tpu_sc as plsc

N = 10000
K = 32
NE = N * K
H = 128
XE = 128

NW = 32
NPW = 320
CH = 64
NCHUNK = NPW // CH
N_PAD = NW * NPW
NE_PAD = N_PAD * K
BLK = 128
SUB = BLK // 16
NB = CH * K // BLK
NBUF = 2


def _sc_body(h_hbm, c_hbm, type_hbm, src_hbm, zeros_hbm,
             mail_hbm, cout_hbm, tout_hbm,
             typev, idx_all, tbuf_all, didx, idxb, hrows, crows, accsh,
             semg, semsc, semcw):
    sid = lax.axis_index("s")
    wid = lax.axis_index("c") * 16 + sid
    abase = sid * (2 * CH)
    pltpu.sync_copy(type_hbm, typev)
    iota16 = lax.iota(jnp.int32, 16)

    def _chunk(ci, carry):
        node_start = wid * NPW + ci * CH
        e0c = node_start * K
        pltpu.sync_copy(src_hbm.at[pl.ds(e0c, CH * K)], idx_all)
        pltpu.sync_copy(zeros_hbm, accsh.at[pl.ds(abase, 2 * CH)])

        gh = [None] * NB
        gc = [None] * NB
        outs = [None] * NB

        def compute_idx(bi):
            b = bi % NBUF
            for j in range(SUB):
                base = bi * BLK + j * 16
                src16 = idx_all[pl.ds(base, 16)]
                idxb[b][pl.ds(j * 16, 16)] = src16
                r16 = lax.shift_right_logical(src16, 7)
                c16 = jnp.bitwise_and(src16, 127)
                t16 = plsc.load_gather(typev, [r16, c16])
                tbuf_all[pl.ds(base, 16)] = t16
                nl = lax.shift_right_logical(base + iota16, 5)
                didx[b][pl.ds(j * 16, 16)] = abase + nl * 2 + t16

        def launch_gathers(bi):
            b = bi % NBUF
            gh[bi] = pltpu.async_copy(h_hbm.at[idxb[b]], hrows[b], semg[b])
            gc[bi] = pltpu.async_copy(c_hbm.at[idxb[b]], crows[b], semg[b])

        def launch_outputs(bi):
            b = bi % NBUF
            o2 = pltpu.async_copy(crows[b],
                                  cout_hbm.at[pl.ds(e0c + bi * BLK, BLK)],
                                  semcw[b])
            pltpu.sync_copy(hrows[b], accsh.at[didx[b]], add=True)
            outs[bi] = (o2,)

        for bi in range(NB + 1):
            if bi < NB:
                if bi >= NBUF:
                    for o in outs[bi - NBUF]:
                        o.wait()
                compute_idx(bi)
                launch_gathers(bi)
            if bi >= 1:
                gh[bi - 1].wait()
                gc[bi - 1].wait()
                launch_outputs(bi - 1)
        for bi in range(NB - NBUF, NB):
            for o in outs[bi]:
                o.wait()

        pltpu.sync_copy(tbuf_all, tout_hbm.at[pl.ds(e0c, CH * K)])
        pltpu.sync_copy(accsh.at[pl.ds(abase, 2 * CH)],
                        mail_hbm.at[pl.ds(node_start * 2, 2 * CH)])
        return carry

    lax.fori_loop(0, NCHUNK, _chunk, jnp.int32(0))


_sc_gather = pl.kernel(
    _sc_body,
    out_type=(
        jax.ShapeDtypeStruct((2 * N_PAD, H), jnp.float32),
        jax.ShapeDtypeStruct((NE_PAD, H), jnp.float32),
        jax.ShapeDtypeStruct((NE_PAD,), jnp.int32),
    ),
    mesh=plsc.VectorSubcoreMesh(core_axis_name="c", subcore_axis_name="s"),
    scratch_types=[
        pltpu.VMEM((80, 128), jnp.int32),
        pltpu.VMEM((CH * K,), jnp.int32),
        pltpu.VMEM((CH * K,), jnp.int32),
        [pltpu.VMEM((BLK,), jnp.int32) for _ in range(NBUF)],
        [pltpu.VMEM((BLK,), jnp.int32) for _ in range(NBUF)],
        [pltpu.VMEM((BLK, H), jnp.float32) for _ in range(NBUF)],
        [pltpu.VMEM((BLK, H), jnp.float32) for _ in range(NBUF)],
        pltpu.VMEM_SHARED((16 * 2 * CH, H), jnp.float32),
        [pltpu.SemaphoreType.DMA for _ in range(NBUF)],
        [pltpu.SemaphoreType.DMA for _ in range(NBUF)],
        [pltpu.SemaphoreType.DMA for _ in range(NBUF)],
    ],
    compiler_params=pltpu.CompilerParams(needs_layout_passes=False),
)

BN = 400
BE = BN * K
GRID = N // BN


def _tc_body(emb3_ref, mail_ref, crows_ref, t_ref,
             W_iou_ref, U_iou_ref, b_iou_ref, W_f_ref, U_f_ref, U_f_b_ref,
             b_f_ref, h_out, c_out):
    i = pl.program_id(0)

    def matT(x, w):
        return lax.dot_general(x, w, (((1,), (1,)), ((), ())),
                               preferred_element_type=jnp.float32)

    h_iou = mail_ref[...]
    f = matT(h_iou, U_f_ref[...]) + U_f_b_ref[...]
    b_f = b_f_ref[...]
    f0 = f[:, :H] + b_f
    f1 = f[:, H:] + b_f

    emb_blk = emb3_ref[pl.ds(i * BN, BN), :]
    iou = (matT(emb_blk, W_iou_ref[...]) + matT(h_iou, U_iou_ref[...])
           + b_iou_ref[...])

    s = (i * BE) % N
    embe = emb3_ref[pl.ds(s, BE), :]
    Xe = matT(embe, W_f_ref[...]).reshape(BN, K, H)
    tb = lax.broadcast_in_dim(t_ref[...].astype(jnp.float32),
                              (BN, K, H), (0, 1))
    f0b = lax.broadcast_in_dim(f0, (BN, K, H), (0, 2))
    dfb = lax.broadcast_in_dim(f1 - f0, (BN, K, H), (0, 2))
    w = jax.nn.sigmoid(Xe + f0b + tb * dfb)
    c_cell = jnp.sum(w * crows_ref[...].reshape(BN, K, H), axis=1)

    ig = jax.nn.sigmoid(iou[:, :H])
    og = jax.nn.sigmoid(iou[:, H:2 * H])
    ug = jnp.tanh(iou[:, 2 * H:])
    c_new = ig * ug + c_cell
    h_out[...] = og * jnp.tanh(c_new)
    c_out[...] = c_new


_tc_dense = pl.pallas_call(
    _tc_body,
    grid=(GRID,),
    in_specs=[
        pl.BlockSpec((3 * N, XE), lambda i: (0, 0)),
        pl.BlockSpec((BN, 2 * H), lambda i: (i, 0)),
        pl.BlockSpec((BE, H), lambda i: (i, 0)),
        pl.BlockSpec((BN, K), lambda i: (i, 0)),
        pl.BlockSpec((3 * H, XE), lambda i: (0, 0)),
        pl.BlockSpec((3 * H, 2 * H), lambda i: (0, 0)),
        pl.BlockSpec((1, 3 * H), lambda i: (0, 0)),
        pl.BlockSpec((H, XE), lambda i: (0, 0)),
        pl.BlockSpec((2 * H, 2 * H), lambda i: (0, 0)),
        pl.BlockSpec((1, 2 * H), lambda i: (0, 0)),
        pl.BlockSpec((1, H), lambda i: (0, 0)),
    ],
    out_specs=[
        pl.BlockSpec((BN, H), lambda i: (i, 0)),
        pl.BlockSpec((BN, H), lambda i: (i, 0)),
    ],
    out_shape=[
        jax.ShapeDtypeStruct((N, H), jnp.float32),
        jax.ShapeDtypeStruct((N, H), jnp.float32),
    ],
)


def kernel(emb, h, c, type_n, edge_index, W_iou_w, U_iou_w, b_iou, W_f_w,
           U_f_w, U_f_b, b_f):
    src = edge_index[0]
    src_pad = jnp.concatenate(
        [src, (jnp.arange(NE_PAD - NE, dtype=src.dtype) * 53) % N])
    zeros = jnp.zeros((2 * CH, H), jnp.float32)
    type_pad = jnp.concatenate(
        [type_n, jnp.zeros((80 * 128 - N,), jnp.int32)]).reshape(80, 128)
    mail, c_rows, t_child = _sc_gather(h, c, type_pad, src_pad, zeros)
    mail2 = mail.reshape(N_PAD, 2 * H)
    t_nk = t_child[:NE].reshape(N, K)
    emb3 = jnp.concatenate([emb, emb, emb], axis=0)
    h_new, c_new = _tc_dense(emb3, mail2, c_rows, t_nk,
                             W_iou_w, U_iou_w, b_iou.reshape(1, 3 * H),
                             W_f_w, U_f_w, U_f_b.reshape(1, 2 * H),
                             b_f.reshape(1, H))
    return (h_new, c_new)


# speedup vs baseline: 2.6664x; 1.0012x over previous
import functools

import jax
import jax.numpy as jnp
from jax import lax
from jax.experimental import pallas as pl
from jax.experimental.pallas import tpu as pltpu
from jax.experimental.pallas import tpu_sc as plsc

N = 10000
K = 32
NE = N * K
H = 128
XE = 128

NW = 32
NPW = 320
CH = 64
NCHUNK = NPW // CH
N_PAD = NW * NPW
NE_PAD = N_PAD * K
BLK = 128
SUB = BLK // 16
NB = CH * K // BLK
NBUF = 2


def _sc_body(h_hbm, c_hbm, type_hbm, src_hbm, zeros_hbm,
             mail_hbm, cout_hbm, tout_hbm,
             typev, idx_all, tbuf_all, didx, idxb, hrows, crows, accsh,
             semg, semsc, semcw):
    sid = lax.axis_index("s")
    wid = lax.axis_index("c") * 16 + sid
    abase = sid * (2 * CH)
    pltpu.sync_copy(type_hbm, typev)
    iota16 = lax.iota(jnp.int32, 16)

    def _chunk(ci, carry):
        node_start = wid * NPW + ci * CH
        e0c = node_start * K
        pltpu.sync_copy(src_hbm.at[pl.ds(e0c, CH * K)], idx_all)
        pltpu.sync_copy(zeros_hbm, accsh.at[pl.ds(abase, 2 * CH)])

        gh = [None] * NB
        gc = [None] * NB
        outs = [None] * NB

        def compute_idx(bi):
            b = bi % NBUF
            for j in range(SUB):
                base = bi * BLK + j * 16
                src16 = idx_all[pl.ds(base, 16)]
                idxb[b][pl.ds(j * 16, 16)] = src16
                r16 = lax.shift_right_logical(src16, 7)
                c16 = jnp.bitwise_and(src16, 127)
                t16 = plsc.load_gather(typev, [r16, c16])
                tbuf_all[pl.ds(base, 16)] = t16
                nl = lax.shift_right_logical(base + iota16, 5)
                didx[b][pl.ds(j * 16, 16)] = abase + nl * 2 + t16

        def launch_gathers(bi):
            b = bi % NBUF
            gh[bi] = pltpu.async_copy(h_hbm.at[idxb[b]], hrows[b], semg[b])
            gc[bi] = pltpu.async_copy(c_hbm.at[idxb[b]], crows[b], semg[b])

        def launch_outputs(bi):
            b = bi % NBUF
            o1 = pltpu.async_copy(hrows[b], accsh.at[didx[b]], semsc[b],
                                  add=True)
            o2 = pltpu.async_copy(crows[b],
                                  cout_hbm.at[pl.ds(e0c + bi * BLK, BLK)],
                                  semcw[b])
            outs[bi] = (o1, o2)

        for bi in range(NB + 1):
            if bi < NB:
                if bi >= NBUF:
                    for o in outs[bi - NBUF]:
                        o.wait()
                compute_idx(bi)
                launch_gathers(bi)
            if bi >= 1:
                gh[bi - 1].wait()
                gc[bi - 1].wait()
                launch_outputs(bi - 1)
        for bi in range(NB - NBUF, NB):
            for o in outs[bi]:
                o.wait()

        pltpu.sync_copy(tbuf_all, tout_hbm.at[pl.ds(e0c, CH * K)])
        pltpu.sync_copy(accsh.at[pl.ds(abase, 2 * CH)],
                        mail_hbm.at[pl.ds(node_start * 2, 2 * CH)])
        return carry

    lax.fori_loop(0, NCHUNK, _chunk, jnp.int32(0))


_sc_gather = pl.kernel(
    _sc_body,
    out_type=(
        jax.ShapeDtypeStruct((2 * N_PAD, H), jnp.float32),
        jax.ShapeDtypeStruct((NE_PAD, H), jnp.float32),
        jax.ShapeDtypeStruct((NE_PAD,), jnp.int32),
    ),
    mesh=plsc.VectorSubcoreMesh(core_axis_name="c", subcore_axis_name="s"),
    scratch_types=[
        pltpu.VMEM((80, 128), jnp.int32),
        pltpu.VMEM((CH * K,), jnp.int32),
        pltpu.VMEM((CH * K,), jnp.int32),
        [pltpu.VMEM((BLK,), jnp.int32) for _ in range(NBUF)],
        [pltpu.VMEM((BLK,), jnp.int32) for _ in range(NBUF)],
        [pltpu.VMEM((BLK, H), jnp.float32) for _ in range(NBUF)],
        [pltpu.VMEM((BLK, H), jnp.float32) for _ in range(NBUF)],
        pltpu.VMEM_SHARED((16 * 2 * CH, H), jnp.float32),
        [pltpu.SemaphoreType.DMA for _ in range(NBUF)],
        [pltpu.SemaphoreType.DMA for _ in range(NBUF)],
        [pltpu.SemaphoreType.DMA for _ in range(NBUF)],
    ],
    compiler_params=pltpu.CompilerParams(needs_layout_passes=False),
)

BN = 400
BE = BN * K
GRID = N // BN


def _tc_body(emb3_ref, mail_ref, crows_ref, t_ref,
             W_iou_ref, U_iou_ref, b_iou_ref, W_f_ref, U_f_ref, U_f_b_ref,
             b_f_ref, h_out, c_out):
    i = pl.program_id(0)

    def matT(x, w):
        return lax.dot_general(x, w, (((1,), (1,)), ((), ())),
                               preferred_element_type=jnp.float32)

    h_iou = mail_ref[...]
    f = matT(h_iou, U_f_ref[...]) + U_f_b_ref[...]
    b_f = b_f_ref[...]
    f0 = f[:, :H] + b_f
    f1 = f[:, H:] + b_f

    emb_blk = emb3_ref[pl.ds(i * BN, BN), :]
    iou = (matT(emb_blk, W_iou_ref[...]) + matT(h_iou, U_iou_ref[...])
           + b_iou_ref[...])

    s = (i * BE) % N
    embe = emb3_ref[pl.ds(s, BE), :]
    Xe = matT(embe, W_f_ref[...]).reshape(BN, K, H)
    tb = lax.broadcast_in_dim(t_ref[...].astype(jnp.float32),
                              (BN, K, H), (0, 1))
    f0b = lax.broadcast_in_dim(f0, (BN, K, H), (0, 2))
    dfb = lax.broadcast_in_dim(f1 - f0, (BN, K, H), (0, 2))
    w = jax.nn.sigmoid(Xe + f0b + tb * dfb)
    c_cell = jnp.sum(w * crows_ref[...].reshape(BN, K, H), axis=1)

    ig = jax.nn.sigmoid(iou[:, :H])
    og = jax.nn.sigmoid(iou[:, H:2 * H])
    ug = jnp.tanh(iou[:, 2 * H:])
    c_new = ig * ug + c_cell
    h_out[...] = og * jnp.tanh(c_new)
    c_out[...] = c_new


_tc_dense = pl.pallas_call(
    _tc_body,
    grid=(GRID,),
    in_specs=[
        pl.BlockSpec((3 * N, XE), lambda i: (0, 0)),
        pl.BlockSpec((BN, 2 * H), lambda i: (i, 0)),
        pl.BlockSpec((BE, H), lambda i: (i, 0)),
        pl.BlockSpec((BN, K), lambda i: (i, 0)),
        pl.BlockSpec((3 * H, XE), lambda i: (0, 0)),
        pl.BlockSpec((3 * H, 2 * H), lambda i: (0, 0)),
        pl.BlockSpec((1, 3 * H), lambda i: (0, 0)),
        pl.BlockSpec((H, XE), lambda i: (0, 0)),
        pl.BlockSpec((2 * H, 2 * H), lambda i: (0, 0)),
        pl.BlockSpec((1, 2 * H), lambda i: (0, 0)),
        pl.BlockSpec((1, H), lambda i: (0, 0)),
    ],
    out_specs=[
        pl.BlockSpec((BN, H), lambda i: (i, 0)),
        pl.BlockSpec((BN, H), lambda i: (i, 0)),
    ],
    out_shape=[
        jax.ShapeDtypeStruct((N, H), jnp.float32),
        jax.ShapeDtypeStruct((N, H), jnp.float32),
    ],
)


def kernel(emb, h, c, type_n, edge_index, W_iou_w, U_iou_w, b_iou, W_f_w,
           U_f_w, U_f_b, b_f):
    src = edge_index[0]
    src_pad = jnp.concatenate(
        [src, (jnp.arange(NE_PAD - NE, dtype=src.dtype) * 53) % N])
    zeros = jnp.zeros((2 * CH, H), jnp.float32)
    type_pad = jnp.concatenate(
        [type_n, jnp.zeros((80 * 128 - N,), jnp.int32)]).reshape(80, 128)
    mail, c_rows, t_child = _sc_gather(h, c, type_pad, src_pad, zeros)
    mail2 = mail.reshape(N_PAD, 2 * H)
    t_nk = t_child[:NE].reshape(N, K)
    emb3 = jnp.concatenate([emb, emb, emb], axis=0)
    h_new, c_new = _tc_dense(emb3, mail2, c_rows, t_nk,
                             W_iou_w, U_iou_w, b_iou.reshape(1, 3 * H),
                             W_f_w, U_f_w, U_f_b.reshape(1, 2 * H),
                             b_f.reshape(1, H))
    return (h_new, c_new)
